# baseline (speedup 1.0000x reference)
import functools

import jax
import jax.numpy as jnp
from jax import lax
from jax.experimental import pallas as pl
from jax.experimental.pallas import tpu as pltpu
from jax.experimental.pallas import tpu_sc as plsc

N = 1000
BATCH = 4096
BB = 512
G = BATCH // BB
C = 8
SZ = BB // C
ROWS = 2
NBUF = 2


def _tc_body(shbm, p1_ref, p21_ref, idx_out, part_out, buf, sems, t_scr):
    step = pl.program_id(0)

    def _chunk_copy(slot, blkidx, c):
        return pltpu.make_async_copy(
            shbm.at[pl.ds(blkidx * BB + c * SZ, SZ), pl.ds(0, ROWS), :],
            buf.at[slot, pl.ds(c * SZ, SZ)],
            sems.at[slot, c],
        )

    def _issue(slot, blkidx):
        for c in range(C):
            _chunk_copy(slot, blkidx, c).start()

    @pl.when(step == 0)
    def _():
        _issue(0, 0)
        p21 = p21_ref[...]
        m = jnp.max(p21, axis=0, keepdims=True)
        lse2 = m[0, :] + jnp.log(jnp.sum(jnp.exp(p21 - m), axis=0))
        p1 = p1_ref[0, :]
        m1 = jnp.max(p1)
        lse1 = m1 + jnp.log(jnp.sum(jnp.exp(p1 - m1)))
        t_scr[0, :] = p1 - lse1 - lse2

    @pl.when(step + 1 < G)
    def _():
        _issue((step + 1) % NBUF, step + 1)

    slot = step % NBUF
    for c in range(C):
        _chunk_copy(slot, step, c).wait()

    blk = buf[slot]
    iota3 = lax.broadcasted_iota(jnp.int32, blk.shape, 2)
    mx = jnp.max(blk, axis=2, keepdims=True)
    am = jnp.min(jnp.where(blk == mx, iota3, N), axis=2)
    n1 = am[:, 0]
    n2 = am[:, 1]

    t = t_scr[0, :]
    iota2 = lax.broadcasted_iota(jnp.int32, (BB, N), 1)
    part = jnp.sum(jnp.where(iota2 == n1[:, None], t[None, :], 0.0), axis=1)
    idx_out[0, 0, :] = n2 * N + n1
    part_out[0, 0, :] = part


def _tc_stage(samples, p1_2d, P_2_1):
    return pl.pallas_call(
        _tc_body,
        grid=(G,),
        in_specs=[
            pl.BlockSpec(memory_space=pl.ANY),
            pl.BlockSpec((1, N), lambda i: (0, 0)),
            pl.BlockSpec((N, N), lambda i: (0, 0)),
        ],
        out_specs=[
            pl.BlockSpec((1, 1, BB), lambda i: (i, 0, 0)),
            pl.BlockSpec((1, 1, BB), lambda i: (i, 0, 0)),
        ],
        out_shape=[
            jax.ShapeDtypeStruct((G, 1, BB), jnp.int32),
            jax.ShapeDtypeStruct((G, 1, BB), jnp.float32),
        ],
        scratch_shapes=[
            pltpu.VMEM((NBUF, BB, ROWS, N), jnp.float32),
            pltpu.SemaphoreType.DMA((NBUF, C)),
            pltpu.VMEM((1, N), jnp.float32),
        ],
    )(samples, p1_2d, P_2_1)


def _make_sc_gather():
    mesh = plsc.VectorSubcoreMesh(core_axis_name="c", subcore_axis_name="s")
    NW = 32
    CHUNK = BATCH // NW

    @functools.partial(
        pl.kernel,
        mesh=mesh,
        out_type=jax.ShapeDtypeStruct((BATCH,), jnp.float32),
        scratch_types=[
            pltpu.VMEM((CHUNK,), jnp.int32),
            pltpu.VMEM((CHUNK,), jnp.float32),
            pltpu.VMEM((CHUNK,), jnp.float32),
            pltpu.VMEM((CHUNK,), jnp.float32),
            pltpu.SemaphoreType.DMA,
        ],
    )
    def sc_gather(tab_hbm, idx_hbm, part_hbm, out_hbm,
                  idx_v, val_v, part_v, out_v, sem):
        wid = lax.axis_index("s") * 2 + lax.axis_index("c")
        base = wid * CHUNK
        pltpu.sync_copy(idx_hbm.at[pl.ds(base, CHUNK)], idx_v)
        pltpu.sync_copy(part_hbm.at[pl.ds(base, CHUNK)], part_v)
        pltpu.async_copy(tab_hbm.at[idx_v], val_v, sem).wait()
        for k in range(CHUNK // 16):
            s = pl.ds(k * 16, 16)
            out_v[s] = val_v[s] + part_v[s]
        pltpu.sync_copy(out_v, out_hbm.at[pl.ds(base, CHUNK)])

    return sc_gather


def kernel(samples, P_1, P_2_1):
    p1_2d = P_1.reshape(1, N)
    idx3, part3 = _tc_stage(samples, p1_2d, P_2_1)
    flat_idx = idx3.reshape(BATCH)
    partial = part3.reshape(BATCH)
    tab = P_2_1.reshape(N * N)
    return _make_sc_gather()(tab, flat_idx, partial)
